# baseline (speedup 1.0000x reference)
import functools
import math

import jax
import jax.numpy as jnp
from jax import lax
from jax.experimental import pallas as pl
from jax.experimental.pallas import tpu as pltpu
from jax.experimental.pallas import tpu_sc as plsc

N = 10000
E = 160000
B = 64
S = 75
SP = 80
H = 32
EH = 128
TH = 128
HEADS = 8
HD = TH // HEADS
FC = 64
STEPS = 3
ISQ = 1.0 / math.sqrt(TH)

NC, NS = 2, 16
NW = NC * NS
EPW = E // NW
NPS = N // NS

EA = 96000
EB = E - EA
ET = 1000
NT = 1000
GB = 8
NEG = -1e30

f32 = jnp.float32


def _k_node_proj(x_ref, w_ref, b_ref, o_ref):
    o_ref[...] = jax.nn.relu(
        jnp.dot(x_ref[...], w_ref[...], preferred_element_type=f32)
        + b_ref[...])


def _node_proj(x, w, b):
    return pl.pallas_call(
        _k_node_proj,
        grid=(N // NT,),
        in_specs=[pl.BlockSpec((NT, 128), lambda i: (i, 0)),
                  pl.BlockSpec((128, H), lambda i: (0, 0)),
                  pl.BlockSpec((1, H), lambda i: (0, 0))],
        out_specs=pl.BlockSpec((NT, H), lambda i: (i, 0)),
        out_shape=jax.ShapeDtypeStruct((N, H), f32),
    )(x, w, b.reshape(1, H))


def _k_edge_net(ea_ref, w1_ref, b1_ref, w2_ref, b2_ref, o_ref):
    a = jax.nn.relu(
        jnp.dot(ea_ref[...], w1_ref[...], preferred_element_type=f32)
        + b1_ref[...])
    o_ref[...] = (jnp.dot(a, w2_ref[...], preferred_element_type=f32)
                  + b2_ref[...])


def _edge_net(ea, w1, b1, w2, b2):
    return pl.pallas_call(
        _k_edge_net,
        grid=(E // ET,),
        in_specs=[pl.BlockSpec((ET, 16), lambda i: (i, 0)),
                  pl.BlockSpec((16, EH), lambda i: (0, 0)),
                  pl.BlockSpec((1, EH), lambda i: (0, 0)),
                  pl.BlockSpec((EH, H * H), lambda i: (0, 0)),
                  pl.BlockSpec((1, H * H), lambda i: (0, 0))],
        out_specs=pl.BlockSpec((ET, H * H), lambda i: (i, 0)),
        out_shape=jax.ShapeDtypeStruct((E, H * H), f32),
    )(ea, w1, b1.reshape(1, EH), w2, b2.reshape(1, H * H))


def _k_msg(g_ref, ew_ref, ex_ref, fo_ref, o_ref):
    gexp = jnp.dot(g_ref[...], ex_ref[...], preferred_element_type=f32)
    o_ref[...] = jnp.dot(gexp * ew_ref[...], fo_ref[...],
                         preferred_element_type=f32)


def _msg(g, ew, expand, fold, eoff):
    ne = g.shape[0]
    blk = eoff // ET
    return pl.pallas_call(
        _k_msg,
        grid=(ne // ET,),
        in_specs=[pl.BlockSpec((ET, H), lambda i: (i, 0)),
                  pl.BlockSpec((ET, H * H), lambda i: (i + blk, 0)),
                  pl.BlockSpec((H, H * H), lambda i: (0, 0)),
                  pl.BlockSpec((H * H, H), lambda i: (0, 0))],
        out_specs=pl.BlockSpec((ET, H), lambda i: (i, 0)),
        out_shape=jax.ShapeDtypeStruct((ne, H), f32),
    )(g, ew, expand, fold)


def _k_gru(p0_ref, p1_ref, p2_ref, p3_ref, h_ref, root_ref, nb_ref,
           wir_ref, wiz_ref, win_ref, whr_ref, whz_ref, whn_ref,
           bir_ref, biz_ref, bin_ref, bhr_ref, bhz_ref, bhn_ref, o_ref):
    h = h_ref[...]
    conv = jax.nn.relu(
        (p0_ref[...] + p1_ref[...]) + (p2_ref[...] + p3_ref[...])
        + jnp.dot(h, root_ref[...], preferred_element_type=f32)
        + nb_ref[...])
    ir = jnp.dot(conv, wir_ref[...], preferred_element_type=f32) + bir_ref[...]
    iz = jnp.dot(conv, wiz_ref[...], preferred_element_type=f32) + biz_ref[...]
    inn = jnp.dot(conv, win_ref[...], preferred_element_type=f32) + bin_ref[...]
    hr = jnp.dot(h, whr_ref[...], preferred_element_type=f32) + bhr_ref[...]
    hz = jnp.dot(h, whz_ref[...], preferred_element_type=f32) + bhz_ref[...]
    hn = jnp.dot(h, whn_ref[...], preferred_element_type=f32) + bhn_ref[...]
    r = jax.nn.sigmoid(ir + hr)
    z = jax.nn.sigmoid(iz + hz)
    ng = jnp.tanh(inn + r * hn)
    o_ref[...] = (1.0 - z) * ng + z * h


def _gru(p0, p1, p2, p3, h, root, nb, wi, bi, wh, bh):
    nspec = pl.BlockSpec((NT, H), lambda i: (i, 0))
    wspec = pl.BlockSpec((H, H), lambda i: (0, 0))
    bspec = pl.BlockSpec((1, H), lambda i: (0, 0))
    return pl.pallas_call(
        _k_gru,
        grid=(N // NT,),
        in_specs=[nspec, nspec, nspec, nspec, nspec, wspec, bspec,
                  wspec, wspec, wspec, wspec, wspec, wspec,
                  bspec, bspec, bspec, bspec, bspec, bspec],
        out_specs=nspec,
        out_shape=jax.ShapeDtypeStruct((N, H), f32),
    )(p0, p1, p2, p3, h, root, nb.reshape(1, H),
      wi[0], wi[1], wi[2], wh[0], wh[1], wh[2],
      bi[0], bi[1], bi[2], bh[0], bh[1], bh[2])


def _k_gcnprep(nf_ref, d0_ref, d1_ref, lw_ref, lb_ref, kw_ref, vw_ref,
               xg_ref, hk_ref, hv_ref):
    xg = (jnp.dot(nf_ref[...], lw_ref[...], preferred_element_type=f32)
          + lb_ref[...])
    indeg = jnp.sum(d0_ref[...] + d1_ref[...], axis=1, keepdims=True) / 16.0
    dis = lax.rsqrt(indeg + 1.0)
    xg_ref[...] = xg
    hk_ref[...] = jnp.dot(xg, kw_ref[...], preferred_element_type=f32) * dis
    hv_ref[...] = jnp.dot(xg, vw_ref[...], preferred_element_type=f32) * dis


def _gcnprep(nf, d0, d1, lw, lb, kw, vw):
    nspec = pl.BlockSpec((NT, TH), lambda i: (i, 0))
    return pl.pallas_call(
        _k_gcnprep,
        grid=(N // NT,),
        in_specs=[pl.BlockSpec((NT, H), lambda i: (i, 0)),
                  pl.BlockSpec((NT, 16), lambda i: (i, 0)),
                  pl.BlockSpec((NT, 16), lambda i: (i, 0)),
                  pl.BlockSpec((H, TH), lambda i: (0, 0)),
                  pl.BlockSpec((1, TH), lambda i: (0, 0)),
                  pl.BlockSpec((TH, TH), lambda i: (0, 0)),
                  pl.BlockSpec((TH, TH), lambda i: (0, 0))],
        out_specs=[nspec, nspec, nspec],
        out_shape=[jax.ShapeDtypeStruct((N, TH), f32)] * 3,
    )(nf, d0, d1, lw, lb.reshape(1, TH), kw, vw)


def _k_gcnfin(pk0_ref, pk1_ref, pv0_ref, pv1_ref, hk_ref, hv_ref,
              d0_ref, d1_ref, kb_ref, vb_ref, kx_ref, vx_ref):
    indeg = jnp.sum(d0_ref[...] + d1_ref[...], axis=1, keepdims=True) / 16.0
    dis = lax.rsqrt(indeg + 1.0)
    kx_ref[...] = dis * (pk0_ref[...] + pk1_ref[...] + hk_ref[...]) + kb_ref[...]
    vx_ref[...] = dis * (pv0_ref[...] + pv1_ref[...] + hv_ref[...]) + vb_ref[...]


def _gcnfin(pk0, pk1, pv0, pv1, hk, hv, d0, d1, kb, vb):
    nspec = pl.BlockSpec((NT, TH), lambda i: (i, 0))
    dspec = pl.BlockSpec((NT, 16), lambda i: (i, 0))
    bspec = pl.BlockSpec((1, TH), lambda i: (0, 0))
    return pl.pallas_call(
        _k_gcnfin,
        grid=(N // NT,),
        in_specs=[nspec, nspec, nspec, nspec, nspec, nspec,
                  dspec, dspec, bspec, bspec],
        out_specs=[nspec, nspec],
        out_shape=[jax.ShapeDtypeStruct((N, TH), f32)] * 2,
    )(pk0, pk1, pv0, pv1, hk, hv, d0, d1,
      kb.reshape(1, TH), vb.reshape(1, TH))


def _k_qp(s1_ref, w_ref, b_ref, o_ref):
    o_ref[...] = (jnp.dot(s1_ref[...], w_ref[...], preferred_element_type=f32)
                  + b_ref[...])


def _qp(s1p, w, b):
    return pl.pallas_call(
        _k_qp,
        grid=(1,),
        in_specs=[pl.BlockSpec((SP, TH), lambda i: (0, 0)),
                  pl.BlockSpec((TH, TH), lambda i: (0, 0)),
                  pl.BlockSpec((1, TH), lambda i: (0, 0))],
        out_specs=pl.BlockSpec((SP, TH), lambda i: (0, 0)),
        out_shape=jax.ShapeDtypeStruct((SP, TH), f32),
    )(s1p, w, b.reshape(1, TH))


def _k_pool(kx_ref, vx_ref, bat_ref, qb_ref, sel_ref, r80_ref, t16_ref,
            o_ref):
    i = pl.program_id(0)

    @pl.when(i == 0)
    def _init():
        o_ref[...] = jnp.zeros_like(o_ref)

    kx = kx_ref[...]
    vx = vx_ref[...]
    brow = bat_ref[0]
    bb = jnp.broadcast_to(brow, (B, NT))
    gid = lax.broadcasted_iota(jnp.int32, (B, NT), 0)
    m = jnp.where(gid == bb, 1.0, 0.0)
    smask = lax.broadcasted_iota(jnp.int32, (NT, SP), 1) >= S
    r80 = r80_ref[...]
    t16 = t16_ref[...]
    for h in range(HEADS):
        qbh = qb_ref[h * TH:(h + 1) * TH, :]
        sc = jnp.dot(kx, qbh, preferred_element_type=f32) * ISQ
        sc = jnp.where(smask, NEG, sc)
        mx = jnp.max(sc, axis=1, keepdims=True)
        e = jnp.exp(sc - mx)
        a = e / jnp.sum(e, axis=1, keepdims=True)
        vh = jnp.dot(vx, sel_ref[h * TH:(h + 1) * TH, :],
                     preferred_element_type=f32)
        p = (jnp.dot(a, r80, preferred_element_type=f32)
             * jnp.dot(vh, t16, preferred_element_type=f32))
        o_ref[:, h * (SP * HD):(h + 1) * (SP * HD)] += jnp.dot(
            m, p, preferred_element_type=f32)


def _pool(kx, vx, bat3, qb, sel, r80, t16):
    return pl.pallas_call(
        _k_pool,
        grid=(N // NT,),
        in_specs=[pl.BlockSpec((NT, TH), lambda i: (i, 0)),
                  pl.BlockSpec((NT, TH), lambda i: (i, 0)),
                  pl.BlockSpec((1, 1, NT), lambda i: (i, 0, 0)),
                  pl.BlockSpec((HEADS * TH, SP), lambda i: (0, 0)),
                  pl.BlockSpec((HEADS * TH, HD), lambda i: (0, 0)),
                  pl.BlockSpec((SP, SP * HD), lambda i: (0, 0)),
                  pl.BlockSpec((HD, SP * HD), lambda i: (0, 0))],
        out_specs=pl.BlockSpec((B, HEADS * SP * HD), lambda i: (0, 0)),
        out_shape=jax.ShapeDtypeStruct((B, HEADS * SP * HD), f32),
    )(kx, vx, bat3, qb, sel, r80, t16)


def _k_tail(pool_ref, qp_ref, sel_ref, selt_ref,
            p1ow_ref, p1ob_ref,
            p2qw_ref, p2qb_ref, p2kw_ref, p2kb_ref, p2vw_ref, p2vb_ref,
            p2ow_ref, p2ob_ref, s3_ref, p3qw_ref, p3qb_ref,
            p3vw_ref, p3vb_ref, p3ow_ref, p3ob_ref,
            l2w_ref, l2b_ref, f0w_ref, f0b_ref, ow_ref, ob_ref, o_ref):
    def mm(a, w, bias):
        return jnp.dot(a, w[...], preferred_element_type=f32) + bias[...]

    ys = []
    for gb in range(GB):
        ys.append(_tail_one(pool_ref[gb * SP:(gb + 1) * SP, :] + qp_ref[...],
                            mm, sel_ref, selt_ref, p1ow_ref, p1ob_ref,
                            p2qw_ref, p2qb_ref, p2kw_ref, p2kb_ref,
                            p2vw_ref, p2vb_ref, p2ow_ref, p2ob_ref,
                            s3_ref, p3qw_ref, p3qb_ref, p3vw_ref, p3vb_ref,
                            p3ow_ref, p3ob_ref, l2w_ref, l2b_ref,
                            f0w_ref, f0b_ref, ow_ref, ob_ref))
    o_ref[...] = jnp.concatenate(ys, axis=0)


def _tail_one(o1, mm, sel_ref, selt_ref, p1ow_ref, p1ob_ref,
              p2qw_ref, p2qb_ref, p2kw_ref, p2kb_ref, p2vw_ref, p2vb_ref,
              p2ow_ref, p2ob_ref, s3_ref, p3qw_ref, p3qb_ref,
              p3vw_ref, p3vb_ref, p3ow_ref, p3ob_ref,
              l2w_ref, l2b_ref, f0w_ref, f0b_ref, ow_ref, ob_ref):
    o1 = o1 + jax.nn.relu(mm(o1, p1ow_ref, p1ob_ref))
    q2 = mm(o1, p2qw_ref, p2qb_ref)
    k2 = mm(o1, p2kw_ref, p2kb_ref)
    v2 = mm(o1, p2vw_ref, p2vb_ref)
    rmask = lax.broadcasted_iota(jnp.int32, (SP, SP), 0) >= S
    cmask = lax.broadcasted_iota(jnp.int32, (SP, SP), 1) >= S
    o2 = jnp.zeros((SP, TH), f32)
    for h in range(HEADS):
        selh = sel_ref[h * TH:(h + 1) * TH, :]
        qh = jnp.dot(q2, selh, preferred_element_type=f32)
        kh = jnp.dot(k2, selh, preferred_element_type=f32)
        vh = jnp.dot(v2, selh, preferred_element_type=f32)
        sc = lax.dot_general(qh, kh, (((1,), (1,)), ((), ())),
                             preferred_element_type=f32) * ISQ
        sc = jnp.where(rmask, NEG, sc)
        mx = jnp.max(sc, axis=0, keepdims=True)
        e2 = jnp.exp(sc - mx)
        a2 = e2 / jnp.sum(e2, axis=0, keepdims=True)
        a2 = jnp.where(cmask, 0.0, a2)
        oh = qh + jnp.dot(a2, vh, preferred_element_type=f32)
        o2 = o2 + jnp.dot(oh, selt_ref[h * HD:(h + 1) * HD, :],
                          preferred_element_type=f32)
    o2 = o2 + jax.nn.relu(mm(o2, p2ow_ref, p2ob_ref))
    qp3 = mm(s3_ref[...], p3qw_ref, p3qb_ref)
    v3 = mm(o2, p3vw_ref, p3vb_ref)
    vmask = lax.broadcasted_iota(jnp.int32, (SP, TH), 0) >= S
    v3 = jnp.where(vmask, 0.0, v3)
    o3 = qp3 + jnp.sum(v3, axis=0, keepdims=True)
    o3 = o3 + jax.nn.relu(mm(o3, p3ow_ref, p3ob_ref))
    y = mm(o3, l2w_ref, l2b_ref)
    y = jax.nn.relu(mm(y, f0w_ref, f0b_ref))
    return mm(y, ow_ref, ob_ref)


def _tail(poolbsd, qp, sel, selt, p):
    full = lambda r, c: pl.BlockSpec((r, c), lambda b: (0, 0))
    return pl.pallas_call(
        _k_tail,
        grid=(B // GB,),
        in_specs=[pl.BlockSpec((GB * SP, TH), lambda b: (b, 0)),
                  full(SP, TH), full(HEADS * TH, HD), full(TH, TH),
                  full(TH, TH), full(1, TH),
                  full(TH, TH), full(1, TH), full(TH, TH), full(1, TH),
                  full(TH, TH), full(1, TH),
                  full(TH, TH), full(1, TH), full(1, TH),
                  full(TH, TH), full(1, TH),
                  full(TH, TH), full(1, TH), full(TH, TH), full(1, TH),
                  full(TH, FC), full(1, FC), full(FC, FC), full(1, FC),
                  full(FC, TH), full(1, TH)],
        out_specs=pl.BlockSpec((GB, TH), lambda b: (b, 0)),
        out_shape=jax.ShapeDtypeStruct((B, TH), f32),
    )(poolbsd, qp, sel, selt,
      p['p1o_W'], p['p1o_b'].reshape(1, TH),
      p['p2q_W'], p['p2q_b'].reshape(1, TH),
      p['p2k_W'], p['p2k_b'].reshape(1, TH),
      p['p2v_W'], p['p2v_b'].reshape(1, TH),
      p['p2o_W'], p['p2o_b'].reshape(1, TH),
      p['S3'].reshape(1, TH),
      p['p3q_W'], p['p3q_b'].reshape(1, TH),
      p['p3v_W'], p['p3v_b'].reshape(1, TH),
      p['p3o_W'], p['p3o_b'].reshape(1, TH),
      p['lin2_W'], p['lin2_b'].reshape(1, FC),
      p['fc0_W'], p['fc0_b'].reshape(1, FC),
      jnp.pad(p['out_W'], ((0, 0), (0, TH - 1))),
      jnp.pad(p['out_b'], (0, TH - 1)).reshape(1, TH))


_MESH = plsc.VectorSubcoreMesh(core_axis_name="c", subcore_axis_name="s")


def _sc_gather(table, idx, ioff, epw):
    W = table.shape[1]
    CH = 1000
    NCHUNK = epw // CH
    ne = epw * NW

    @functools.partial(
        pl.kernel, mesh=_MESH,
        compiler_params=pltpu.CompilerParams(use_tc_tiling_on_sc=False),
        out_type=jax.ShapeDtypeStruct((ne, W), f32),
        scratch_types=[pltpu.VMEM((CH,), jnp.int32),
                       pltpu.VMEM((CH, W), f32),
                       pltpu.SemaphoreType.DMA],
    )
    def k(table_hbm, idx_hbm, out_hbm, idx_v, rows_v, sem):
        wid = lax.axis_index("s") * NC + lax.axis_index("c")
        base = wid * epw

        def body(c, _):
            off = base + c * CH
            pltpu.sync_copy(idx_hbm.at[pl.ds(ioff + off, CH)], idx_v)
            pltpu.async_copy(table_hbm.at[idx_v], rows_v, sem).wait()
            pltpu.sync_copy(rows_v, out_hbm.at[pl.ds(off, CH)])
            return _

        lax.fori_loop(0, NCHUNK, body, 0)

    return k(table, idx)


def _sc_scatter_add(rows, idx, width, chunk, ioff, epw):
    W = width
    CH = chunk
    NCHUNK = epw // CH

    @functools.partial(
        pl.kernel, mesh=_MESH,
        compiler_params=pltpu.CompilerParams(use_tc_tiling_on_sc=False),
        out_type=jax.ShapeDtypeStruct((NC * N, W), f32),
        scratch_types=[pltpu.VMEM((CH,), jnp.int32),
                       pltpu.VMEM((CH, W), f32),
                       pltpu.VMEM_SHARED((N, W), f32)],
    )
    def k(rows_hbm, idx_hbm, zero_hbm, out_hbm, idx_v, rows_v, acc):
        cid = lax.axis_index("c")
        sid = lax.axis_index("s")
        wid = sid * NC + cid
        base = wid * epw
        pltpu.sync_copy(zero_hbm.at[pl.ds(sid * NPS, NPS)],
                        acc.at[pl.ds(sid * NPS, NPS)])
        plsc.subcore_barrier()

        def body(c, _):
            off = base + c * CH
            pltpu.sync_copy(idx_hbm.at[pl.ds(ioff + off, CH)], idx_v)
            pltpu.sync_copy(rows_hbm.at[pl.ds(off, CH)], rows_v)
            pltpu.sync_copy(rows_v, acc.at[idx_v], add=True)
            return _

        lax.fori_loop(0, NCHUNK, body, 0)
        plsc.subcore_barrier()
        pltpu.sync_copy(acc.at[pl.ds(sid * NPS, NPS)],
                        out_hbm.at[pl.ds(cid * N + sid * NPS, NPS)])

    return k(rows, idx, jnp.zeros((N, W), f32))


def _sc_degree(idx):
    W = 16
    CH = 1000
    NCHUNK = EPW // CH

    @functools.partial(
        pl.kernel, mesh=_MESH,
        compiler_params=pltpu.CompilerParams(use_tc_tiling_on_sc=False),
        out_type=jax.ShapeDtypeStruct((NC * N, W), f32),
        scratch_types=[pltpu.VMEM((CH,), jnp.int32),
                       pltpu.VMEM((CH, W), f32),
                       pltpu.VMEM_SHARED((N, W), f32)],
    )
    def k(idx_hbm, one_hbm, zero_hbm, out_hbm, idx_v, ones_v, acc):
        cid = lax.axis_index("c")
        sid = lax.axis_index("s")
        wid = sid * NC + cid
        base = wid * EPW
        pltpu.sync_copy(zero_hbm.at[pl.ds(sid * NPS, NPS)],
                        acc.at[pl.ds(sid * NPS, NPS)])
        pltpu.sync_copy(one_hbm, ones_v)
        plsc.subcore_barrier()

        def body(c, _):
            pltpu.sync_copy(idx_hbm.at[pl.ds(base + c * CH, CH)], idx_v)
            pltpu.sync_copy(ones_v, acc.at[idx_v], add=True)
            return _

        lax.fori_loop(0, NCHUNK, body, 0)
        plsc.subcore_barrier()
        pltpu.sync_copy(acc.at[pl.ds(sid * NPS, NPS)],
                        out_hbm.at[pl.ds(cid * N + sid * NPS, NPS)])

    return k(idx, jnp.ones((CH, W), f32), jnp.zeros((N, W), f32))


def _sc_gcn_pass(table, src, dst):
    W = TH
    CH = 200
    NCHUNK = EPW // CH

    @functools.partial(
        pl.kernel, mesh=_MESH,
        compiler_params=pltpu.CompilerParams(use_tc_tiling_on_sc=False),
        out_type=jax.ShapeDtypeStruct((NC * N, W), f32),
        scratch_types=[pltpu.VMEM((CH,), jnp.int32),
                       pltpu.VMEM((CH,), jnp.int32),
                       pltpu.VMEM((CH, W), f32),
                       pltpu.VMEM_SHARED((N, W), f32),
                       pltpu.SemaphoreType.DMA],
    )
    def k(table_hbm, src_hbm, dst_hbm, zero_hbm, out_hbm,
          sidx_v, didx_v, rows_v, acc, sem):
        cid = lax.axis_index("c")
        sid = lax.axis_index("s")
        wid = sid * NC + cid
        base = wid * EPW
        pltpu.sync_copy(zero_hbm.at[pl.ds(sid * NPS, NPS)],
                        acc.at[pl.ds(sid * NPS, NPS)])
        plsc.subcore_barrier()

        def body(c, _):
            off = base + c * CH
            pltpu.sync_copy(src_hbm.at[pl.ds(off, CH)], sidx_v)
            pltpu.sync_copy(dst_hbm.at[pl.ds(off, CH)], didx_v)
            pltpu.async_copy(table_hbm.at[sidx_v], rows_v, sem).wait()
            pltpu.sync_copy(rows_v, acc.at[didx_v], add=True)
            return _

        lax.fori_loop(0, NCHUNK, body, 0)
        plsc.subcore_barrier()
        pltpu.sync_copy(acc.at[pl.ds(sid * NPS, NPS)],
                        out_hbm.at[pl.ds(cid * N + sid * NPS, NPS)])

    return k(table, src, dst, jnp.zeros((N, W), f32))


def kernel(x, edge_attr, params, edge_index, batch):
    p = params
    src = edge_index[0]
    dst = edge_index[1]

    eye = jnp.eye(H, dtype=f32)
    expand = jnp.kron(eye, jnp.ones((1, H), f32))
    fold = jnp.tile(eye, (H, 1))

    ew = _edge_net(edge_attr, p['en1_W'], p['en1_b'], p['en2_W'], p['en2_b'])

    h = _node_proj(x, p['proj_W'], p['proj_b'])
    wi = [p['gru_Wih'][i * H:(i + 1) * H].T for i in range(3)]
    bi = [p['gru_bih'][i * H:(i + 1) * H].reshape(1, H) for i in range(3)]
    wh = [p['gru_Whh'][i * H:(i + 1) * H].T for i in range(3)]
    bh = [p['gru_bhh'][i * H:(i + 1) * H].reshape(1, H) for i in range(3)]

    deg = _sc_degree(dst)

    for _ in range(STEPS):
        ga = _sc_gather(h, src, 0, EA // NW)
        gb = _sc_gather(h, src, EA, EB // NW)
        ma = _msg(ga, ew, expand, fold, 0)
        sa = _sc_scatter_add(ma, dst, H, 1000, 0, EA // NW)
        mb = _msg(gb, ew, expand, fold, EA)
        sb = _sc_scatter_add(mb, dst, H, 1000, EA, EB // NW)
        h = _gru(sa[:N], sa[N:], sb[:N], sb[N:], h,
                 p['root'], p['nn_bias'], wi, bi, wh, bh)

    xg, hk, hv = _gcnprep(h, deg[:N], deg[N:], p['lin1_W'], p['lin1_b'],
                          p['p1k_W'], p['p1v_W'])
    pk = _sc_gcn_pass(hk, src, dst)
    pv = _sc_gcn_pass(hv, src, dst)
    kx, vx = _gcnfin(pk[:N], pk[N:], pv[:N], pv[N:], hk, hv, deg[:N], deg[N:],
                     p['p1k_b'], p['p1v_b'])

    s1p = jnp.pad(p['S1'][0], ((0, SP - S), (0, 0)))
    qp = _qp(s1p, p['p1q_W'], p['p1q_b'])

    qsl = qp[:S]
    hmask = jnp.repeat(jnp.eye(HEADS, dtype=f32), HD, axis=0)
    qb = (qsl.T[None, :, :] * hmask.T[:, :, None]).reshape(HEADS * TH, S)
    qb = jnp.pad(qb, ((0, 0), (0, SP - S)))
    sel = jnp.concatenate(
        [jnp.eye(TH, dtype=f32)[:, h * HD:(h + 1) * HD] for h in range(HEADS)],
        axis=0)
    selt = jnp.eye(TH, dtype=f32)
    r80 = jnp.kron(jnp.eye(SP, dtype=f32), jnp.ones((1, HD), f32))
    t16 = jnp.tile(jnp.eye(HD, dtype=f32), (1, SP))

    bat3 = batch.reshape(N // NT, 1, NT)
    pooled = _pool(kx, vx, bat3, qb, sel, r80, t16)

    pooled = pooled.reshape(B, HEADS, SP, HD).transpose(0, 2, 1, 3)
    poolbsd = pooled.reshape(B * SP, TH)

    out = _tail(poolbsd, qp, sel, selt, p)
    return out[:, :1]


# speedup vs baseline: 1.0675x; 1.0675x over previous
import functools
import math

import jax
import jax.numpy as jnp
from jax import lax
from jax.experimental import pallas as pl
from jax.experimental.pallas import tpu as pltpu
from jax.experimental.pallas import tpu_sc as plsc

N = 10000
E = 160000
B = 64
S = 75
SP = 80
H = 32
EH = 128
TH = 128
HEADS = 8
HD = TH // HEADS
FC = 64
STEPS = 3
ISQ = 1.0 / math.sqrt(TH)

NC, NS = 2, 16
NW = NC * NS
EPW = E // NW
NPS = N // NS

EA = 96000
EB = E - EA
ET = 1000
NT = 1000
GB = 8
NEG = -1e30

f32 = jnp.float32


def _k_node_proj(x_ref, w_ref, b_ref, o_ref):
    o_ref[...] = jax.nn.relu(
        jnp.dot(x_ref[...], w_ref[...], preferred_element_type=f32)
        + b_ref[...])


def _node_proj(x, w, b):
    return pl.pallas_call(
        _k_node_proj,
        grid=(N // NT,),
        in_specs=[pl.BlockSpec((NT, 128), lambda i: (i, 0)),
                  pl.BlockSpec((128, H), lambda i: (0, 0)),
                  pl.BlockSpec((1, H), lambda i: (0, 0))],
        out_specs=pl.BlockSpec((NT, H), lambda i: (i, 0)),
        out_shape=jax.ShapeDtypeStruct((N, H), f32),
    )(x, w, b.reshape(1, H))


def _k_edge_net(ea_ref, w1_ref, b1_ref, w2_ref, b2_ref, o_ref, sc_ref):
    a = jax.nn.relu(
        jnp.dot(ea_ref[...], w1_ref[...], preferred_element_type=f32)
        + b1_ref[...])
    ew = (jnp.dot(a, w2_ref[...], preferred_element_type=f32)
          + b2_ref[...])
    mx = jnp.max(jnp.max(jnp.abs(ew), axis=1, keepdims=True),
                 axis=0, keepdims=True)
    mx = jnp.maximum(mx, 1e-20)
    q = jnp.round(ew * (32767.0 / mx))
    o_ref[...] = q.astype(jnp.int32).astype(jnp.int16)
    sc_ref[...] = jnp.broadcast_to(mx / 32767.0, (1, 8, 128))


def _edge_net(ea, w1, b1, w2, b2):
    return pl.pallas_call(
        _k_edge_net,
        grid=(E // ET,),
        in_specs=[pl.BlockSpec((ET, 16), lambda i: (i, 0)),
                  pl.BlockSpec((16, EH), lambda i: (0, 0)),
                  pl.BlockSpec((1, EH), lambda i: (0, 0)),
                  pl.BlockSpec((EH, H * H), lambda i: (0, 0)),
                  pl.BlockSpec((1, H * H), lambda i: (0, 0))],
        out_specs=[pl.BlockSpec((ET, H * H), lambda i: (i, 0)),
                   pl.BlockSpec((1, 8, 128), lambda i: (i, 0, 0))],
        out_shape=[jax.ShapeDtypeStruct((E, H * H), jnp.int16),
                   jax.ShapeDtypeStruct((E // ET, 8, 128), f32)],
    )(ea, w1, b1.reshape(1, EH), w2, b2.reshape(1, H * H))


def _k_msg(g_ref, ew_ref, sc_ref, ex_ref, fo_ref, o_ref):
    gexp = jnp.dot(g_ref[...], ex_ref[...], preferred_element_type=f32)
    ew = ew_ref[...].astype(jnp.int32).astype(f32) * sc_ref[0, 0:1, 0:1]
    o_ref[...] = jnp.dot(gexp * ew, fo_ref[...],
                         preferred_element_type=f32)


def _msg(g, ew, ewsc, expand, fold, eoff):
    ne = g.shape[0]
    blk = eoff // ET
    return pl.pallas_call(
        _k_msg,
        grid=(ne // ET,),
        in_specs=[pl.BlockSpec((ET, H), lambda i: (i, 0)),
                  pl.BlockSpec((ET, H * H), lambda i: (i + blk, 0)),
                  pl.BlockSpec((1, 8, 128), lambda i: (i + blk, 0, 0)),
                  pl.BlockSpec((H, H * H), lambda i: (0, 0)),
                  pl.BlockSpec((H * H, H), lambda i: (0, 0))],
        out_specs=pl.BlockSpec((ET, H), lambda i: (i, 0)),
        out_shape=jax.ShapeDtypeStruct((ne, H), f32),
    )(g, ew, ewsc, expand, fold)


def _k_gru(p0_ref, p1_ref, p2_ref, p3_ref, h_ref, root_ref, nb_ref,
           wir_ref, wiz_ref, win_ref, whr_ref, whz_ref, whn_ref,
           bir_ref, biz_ref, bin_ref, bhr_ref, bhz_ref, bhn_ref, o_ref):
    h = h_ref[...]
    conv = jax.nn.relu(
        (p0_ref[...] + p1_ref[...]) + (p2_ref[...] + p3_ref[...])
        + jnp.dot(h, root_ref[...], preferred_element_type=f32)
        + nb_ref[...])
    ir = jnp.dot(conv, wir_ref[...], preferred_element_type=f32) + bir_ref[...]
    iz = jnp.dot(conv, wiz_ref[...], preferred_element_type=f32) + biz_ref[...]
    inn = jnp.dot(conv, win_ref[...], preferred_element_type=f32) + bin_ref[...]
    hr = jnp.dot(h, whr_ref[...], preferred_element_type=f32) + bhr_ref[...]
    hz = jnp.dot(h, whz_ref[...], preferred_element_type=f32) + bhz_ref[...]
    hn = jnp.dot(h, whn_ref[...], preferred_element_type=f32) + bhn_ref[...]
    r = jax.nn.sigmoid(ir + hr)
    z = jax.nn.sigmoid(iz + hz)
    ng = jnp.tanh(inn + r * hn)
    o_ref[...] = (1.0 - z) * ng + z * h


def _gru(p0, p1, p2, p3, h, root, nb, wi, bi, wh, bh):
    nspec = pl.BlockSpec((NT, H), lambda i: (i, 0))
    wspec = pl.BlockSpec((H, H), lambda i: (0, 0))
    bspec = pl.BlockSpec((1, H), lambda i: (0, 0))
    return pl.pallas_call(
        _k_gru,
        grid=(N // NT,),
        in_specs=[nspec, nspec, nspec, nspec, nspec, wspec, bspec,
                  wspec, wspec, wspec, wspec, wspec, wspec,
                  bspec, bspec, bspec, bspec, bspec, bspec],
        out_specs=nspec,
        out_shape=jax.ShapeDtypeStruct((N, H), f32),
    )(p0, p1, p2, p3, h, root, nb.reshape(1, H),
      wi[0], wi[1], wi[2], wh[0], wh[1], wh[2],
      bi[0], bi[1], bi[2], bh[0], bh[1], bh[2])


def _k_gcnprep(nf_ref, d0_ref, d1_ref, lw_ref, lb_ref, kw_ref, vw_ref,
               xg_ref, hk_ref, hv_ref):
    xg = (jnp.dot(nf_ref[...], lw_ref[...], preferred_element_type=f32)
          + lb_ref[...])
    indeg = jnp.sum(d0_ref[...] + d1_ref[...], axis=1, keepdims=True) / 16.0
    dis = lax.rsqrt(indeg + 1.0)
    xg_ref[...] = xg
    hk_ref[...] = jnp.dot(xg, kw_ref[...], preferred_element_type=f32) * dis
    hv_ref[...] = jnp.dot(xg, vw_ref[...], preferred_element_type=f32) * dis


def _gcnprep(nf, d0, d1, lw, lb, kw, vw):
    nspec = pl.BlockSpec((NT, TH), lambda i: (i, 0))
    return pl.pallas_call(
        _k_gcnprep,
        grid=(N // NT,),
        in_specs=[pl.BlockSpec((NT, H), lambda i: (i, 0)),
                  pl.BlockSpec((NT, 16), lambda i: (i, 0)),
                  pl.BlockSpec((NT, 16), lambda i: (i, 0)),
                  pl.BlockSpec((H, TH), lambda i: (0, 0)),
                  pl.BlockSpec((1, TH), lambda i: (0, 0)),
                  pl.BlockSpec((TH, TH), lambda i: (0, 0)),
                  pl.BlockSpec((TH, TH), lambda i: (0, 0))],
        out_specs=[nspec, nspec, nspec],
        out_shape=[jax.ShapeDtypeStruct((N, TH), f32)] * 3,
    )(nf, d0, d1, lw, lb.reshape(1, TH), kw, vw)


def _k_gcnfin(pk0_ref, pk1_ref, pv0_ref, pv1_ref, hk_ref, hv_ref,
              d0_ref, d1_ref, kb_ref, vb_ref, kx_ref, vx_ref):
    indeg = jnp.sum(d0_ref[...] + d1_ref[...], axis=1, keepdims=True) / 16.0
    dis = lax.rsqrt(indeg + 1.0)
    kx_ref[...] = dis * (pk0_ref[...] + pk1_ref[...] + hk_ref[...]) + kb_ref[...]
    vx_ref[...] = dis * (pv0_ref[...] + pv1_ref[...] + hv_ref[...]) + vb_ref[...]


def _gcnfin(pk0, pk1, pv0, pv1, hk, hv, d0, d1, kb, vb):
    nspec = pl.BlockSpec((NT, TH), lambda i: (i, 0))
    dspec = pl.BlockSpec((NT, 16), lambda i: (i, 0))
    bspec = pl.BlockSpec((1, TH), lambda i: (0, 0))
    return pl.pallas_call(
        _k_gcnfin,
        grid=(N // NT,),
        in_specs=[nspec, nspec, nspec, nspec, nspec, nspec,
                  dspec, dspec, bspec, bspec],
        out_specs=[nspec, nspec],
        out_shape=[jax.ShapeDtypeStruct((N, TH), f32)] * 2,
    )(pk0, pk1, pv0, pv1, hk, hv, d0, d1,
      kb.reshape(1, TH), vb.reshape(1, TH))


def _k_qp(s1_ref, w_ref, b_ref, o_ref):
    o_ref[...] = (jnp.dot(s1_ref[...], w_ref[...], preferred_element_type=f32)
                  + b_ref[...])


def _qp(s1p, w, b):
    return pl.pallas_call(
        _k_qp,
        grid=(1,),
        in_specs=[pl.BlockSpec((SP, TH), lambda i: (0, 0)),
                  pl.BlockSpec((TH, TH), lambda i: (0, 0)),
                  pl.BlockSpec((1, TH), lambda i: (0, 0))],
        out_specs=pl.BlockSpec((SP, TH), lambda i: (0, 0)),
        out_shape=jax.ShapeDtypeStruct((SP, TH), f32),
    )(s1p, w, b.reshape(1, TH))


def _k_pool(kx_ref, vx_ref, bat_ref, qb_ref, sel_ref, r80_ref, t16_ref,
            o_ref):
    i = pl.program_id(0)

    @pl.when(i == 0)
    def _init():
        o_ref[...] = jnp.zeros_like(o_ref)

    kx = kx_ref[...]
    vx = vx_ref[...]
    brow = bat_ref[0]
    bb = jnp.broadcast_to(brow, (B, NT))
    gid = lax.broadcasted_iota(jnp.int32, (B, NT), 0)
    m = jnp.where(gid == bb, 1.0, 0.0)
    smask = lax.broadcasted_iota(jnp.int32, (NT, SP), 1) >= S
    r80 = r80_ref[...]
    t16 = t16_ref[...]
    for h in range(HEADS):
        qbh = qb_ref[h * TH:(h + 1) * TH, :]
        sc = jnp.dot(kx, qbh, preferred_element_type=f32) * ISQ
        sc = jnp.where(smask, NEG, sc)
        mx = jnp.max(sc, axis=1, keepdims=True)
        e = jnp.exp(sc - mx)
        a = e / jnp.sum(e, axis=1, keepdims=True)
        vh = jnp.dot(vx, sel_ref[h * TH:(h + 1) * TH, :],
                     preferred_element_type=f32)
        p = (jnp.dot(a, r80, preferred_element_type=f32)
             * jnp.dot(vh, t16, preferred_element_type=f32))
        o_ref[:, h * (SP * HD):(h + 1) * (SP * HD)] += jnp.dot(
            m, p, preferred_element_type=f32)


def _pool(kx, vx, bat3, qb, sel, r80, t16):
    return pl.pallas_call(
        _k_pool,
        grid=(N // NT,),
        in_specs=[pl.BlockSpec((NT, TH), lambda i: (i, 0)),
                  pl.BlockSpec((NT, TH), lambda i: (i, 0)),
                  pl.BlockSpec((1, 1, NT), lambda i: (i, 0, 0)),
                  pl.BlockSpec((HEADS * TH, SP), lambda i: (0, 0)),
                  pl.BlockSpec((HEADS * TH, HD), lambda i: (0, 0)),
                  pl.BlockSpec((SP, SP * HD), lambda i: (0, 0)),
                  pl.BlockSpec((HD, SP * HD), lambda i: (0, 0))],
        out_specs=pl.BlockSpec((B, HEADS * SP * HD), lambda i: (0, 0)),
        out_shape=jax.ShapeDtypeStruct((B, HEADS * SP * HD), f32),
    )(kx, vx, bat3, qb, sel, r80, t16)


def _k_tail(pool_ref, qp_ref, sel_ref, selt_ref,
            p1ow_ref, p1ob_ref,
            p2qw_ref, p2qb_ref, p2kw_ref, p2kb_ref, p2vw_ref, p2vb_ref,
            p2ow_ref, p2ob_ref, s3_ref, p3qw_ref, p3qb_ref,
            p3vw_ref, p3vb_ref, p3ow_ref, p3ob_ref,
            l2w_ref, l2b_ref, f0w_ref, f0b_ref, ow_ref, ob_ref, o_ref):
    def mm(a, w, bias):
        return jnp.dot(a, w[...], preferred_element_type=f32) + bias[...]

    ys = []
    for gb in range(GB):
        ys.append(_tail_one(pool_ref[gb * SP:(gb + 1) * SP, :] + qp_ref[...],
                            mm, sel_ref, selt_ref, p1ow_ref, p1ob_ref,
                            p2qw_ref, p2qb_ref, p2kw_ref, p2kb_ref,
                            p2vw_ref, p2vb_ref, p2ow_ref, p2ob_ref,
                            s3_ref, p3qw_ref, p3qb_ref, p3vw_ref, p3vb_ref,
                            p3ow_ref, p3ob_ref, l2w_ref, l2b_ref,
                            f0w_ref, f0b_ref, ow_ref, ob_ref))
    o_ref[...] = jnp.concatenate(ys, axis=0)


def _tail_one(o1, mm, sel_ref, selt_ref, p1ow_ref, p1ob_ref,
              p2qw_ref, p2qb_ref, p2kw_ref, p2kb_ref, p2vw_ref, p2vb_ref,
              p2ow_ref, p2ob_ref, s3_ref, p3qw_ref, p3qb_ref,
              p3vw_ref, p3vb_ref, p3ow_ref, p3ob_ref,
              l2w_ref, l2b_ref, f0w_ref, f0b_ref, ow_ref, ob_ref):
    o1 = o1 + jax.nn.relu(mm(o1, p1ow_ref, p1ob_ref))
    q2 = mm(o1, p2qw_ref, p2qb_ref)
    k2 = mm(o1, p2kw_ref, p2kb_ref)
    v2 = mm(o1, p2vw_ref, p2vb_ref)
    rmask = lax.broadcasted_iota(jnp.int32, (SP, SP), 0) >= S
    cmask = lax.broadcasted_iota(jnp.int32, (SP, SP), 1) >= S
    o2 = jnp.zeros((SP, TH), f32)
    for h in range(HEADS):
        selh = sel_ref[h * TH:(h + 1) * TH, :]
        qh = jnp.dot(q2, selh, preferred_element_type=f32)
        kh = jnp.dot(k2, selh, preferred_element_type=f32)
        vh = jnp.dot(v2, selh, preferred_element_type=f32)
        sc = lax.dot_general(qh, kh, (((1,), (1,)), ((), ())),
                             preferred_element_type=f32) * ISQ
        sc = jnp.where(rmask, NEG, sc)
        mx = jnp.max(sc, axis=0, keepdims=True)
        e2 = jnp.exp(sc - mx)
        a2 = e2 / jnp.sum(e2, axis=0, keepdims=True)
        a2 = jnp.where(cmask, 0.0, a2)
        oh = qh + jnp.dot(a2, vh, preferred_element_type=f32)
        o2 = o2 + jnp.dot(oh, selt_ref[h * HD:(h + 1) * HD, :],
                          preferred_element_type=f32)
    o2 = o2 + jax.nn.relu(mm(o2, p2ow_ref, p2ob_ref))
    qp3 = mm(s3_ref[...], p3qw_ref, p3qb_ref)
    v3 = mm(o2, p3vw_ref, p3vb_ref)
    vmask = lax.broadcasted_iota(jnp.int32, (SP, TH), 0) >= S
    v3 = jnp.where(vmask, 0.0, v3)
    o3 = qp3 + jnp.sum(v3, axis=0, keepdims=True)
    o3 = o3 + jax.nn.relu(mm(o3, p3ow_ref, p3ob_ref))
    y = mm(o3, l2w_ref, l2b_ref)
    y = jax.nn.relu(mm(y, f0w_ref, f0b_ref))
    return mm(y, ow_ref, ob_ref)


def _tail(poolbsd, qp, sel, selt, p):
    full = lambda r, c: pl.BlockSpec((r, c), lambda b: (0, 0))
    return pl.pallas_call(
        _k_tail,
        grid=(B // GB,),
        in_specs=[pl.BlockSpec((GB * SP, TH), lambda b: (b, 0)),
                  full(SP, TH), full(HEADS * TH, HD), full(TH, TH),
                  full(TH, TH), full(1, TH),
                  full(TH, TH), full(1, TH), full(TH, TH), full(1, TH),
                  full(TH, TH), full(1, TH),
                  full(TH, TH), full(1, TH), full(1, TH),
                  full(TH, TH), full(1, TH),
                  full(TH, TH), full(1, TH), full(TH, TH), full(1, TH),
                  full(TH, FC), full(1, FC), full(FC, FC), full(1, FC),
                  full(FC, TH), full(1, TH)],
        out_specs=pl.BlockSpec((GB, TH), lambda b: (b, 0)),
        out_shape=jax.ShapeDtypeStruct((B, TH), f32),
    )(poolbsd, qp, sel, selt,
      p['p1o_W'], p['p1o_b'].reshape(1, TH),
      p['p2q_W'], p['p2q_b'].reshape(1, TH),
      p['p2k_W'], p['p2k_b'].reshape(1, TH),
      p['p2v_W'], p['p2v_b'].reshape(1, TH),
      p['p2o_W'], p['p2o_b'].reshape(1, TH),
      p['S3'].reshape(1, TH),
      p['p3q_W'], p['p3q_b'].reshape(1, TH),
      p['p3v_W'], p['p3v_b'].reshape(1, TH),
      p['p3o_W'], p['p3o_b'].reshape(1, TH),
      p['lin2_W'], p['lin2_b'].reshape(1, FC),
      p['fc0_W'], p['fc0_b'].reshape(1, FC),
      jnp.pad(p['out_W'], ((0, 0), (0, TH - 1))),
      jnp.pad(p['out_b'], (0, TH - 1)).reshape(1, TH))


_MESH = plsc.VectorSubcoreMesh(core_axis_name="c", subcore_axis_name="s")


def _sc_gather(table, idx, ioff, epw):
    W = table.shape[1]
    CH = 1000
    NCHUNK = epw // CH
    ne = epw * NW

    @functools.partial(
        pl.kernel, mesh=_MESH,
        compiler_params=pltpu.CompilerParams(use_tc_tiling_on_sc=False),
        out_type=jax.ShapeDtypeStruct((ne, W), f32),
        scratch_types=[pltpu.VMEM((CH,), jnp.int32),
                       pltpu.VMEM((CH, W), f32),
                       pltpu.SemaphoreType.DMA],
    )
    def k(table_hbm, idx_hbm, out_hbm, idx_v, rows_v, sem):
        wid = lax.axis_index("s") * NC + lax.axis_index("c")
        base = wid * epw

        def body(c, _):
            off = base + c * CH
            pltpu.sync_copy(idx_hbm.at[pl.ds(ioff + off, CH)], idx_v)
            pltpu.async_copy(table_hbm.at[idx_v], rows_v, sem).wait()
            pltpu.sync_copy(rows_v, out_hbm.at[pl.ds(off, CH)])
            return _

        lax.fori_loop(0, NCHUNK, body, 0)

    return k(table, idx)


def _sc_scatter_add(rows, idx, width, chunk, ioff, epw):
    W = width
    CH = chunk
    NCHUNK = epw // CH

    @functools.partial(
        pl.kernel, mesh=_MESH,
        compiler_params=pltpu.CompilerParams(use_tc_tiling_on_sc=False),
        out_type=jax.ShapeDtypeStruct((NC * N, W), f32),
        scratch_types=[pltpu.VMEM((CH,), jnp.int32),
                       pltpu.VMEM((CH, W), f32),
                       pltpu.VMEM_SHARED((N, W), f32)],
    )
    def k(rows_hbm, idx_hbm, zero_hbm, out_hbm, idx_v, rows_v, acc):
        cid = lax.axis_index("c")
        sid = lax.axis_index("s")
        wid = sid * NC + cid
        base = wid * epw
        pltpu.sync_copy(zero_hbm.at[pl.ds(sid * NPS, NPS)],
                        acc.at[pl.ds(sid * NPS, NPS)])
        plsc.subcore_barrier()

        def body(c, _):
            off = base + c * CH
            pltpu.sync_copy(idx_hbm.at[pl.ds(ioff + off, CH)], idx_v)
            pltpu.sync_copy(rows_hbm.at[pl.ds(off, CH)], rows_v)
            pltpu.sync_copy(rows_v, acc.at[idx_v], add=True)
            return _

        lax.fori_loop(0, NCHUNK, body, 0)
        plsc.subcore_barrier()
        pltpu.sync_copy(acc.at[pl.ds(sid * NPS, NPS)],
                        out_hbm.at[pl.ds(cid * N + sid * NPS, NPS)])

    return k(rows, idx, jnp.zeros((N, W), f32))


def _sc_degree(idx):
    W = 16
    CH = 1000
    NCHUNK = EPW // CH

    @functools.partial(
        pl.kernel, mesh=_MESH,
        compiler_params=pltpu.CompilerParams(use_tc_tiling_on_sc=False),
        out_type=jax.ShapeDtypeStruct((NC * N, W), f32),
        scratch_types=[pltpu.VMEM((CH,), jnp.int32),
                       pltpu.VMEM((CH, W), f32),
                       pltpu.VMEM_SHARED((N, W), f32)],
    )
    def k(idx_hbm, one_hbm, zero_hbm, out_hbm, idx_v, ones_v, acc):
        cid = lax.axis_index("c")
        sid = lax.axis_index("s")
        wid = sid * NC + cid
        base = wid * EPW
        pltpu.sync_copy(zero_hbm.at[pl.ds(sid * NPS, NPS)],
                        acc.at[pl.ds(sid * NPS, NPS)])
        pltpu.sync_copy(one_hbm, ones_v)
        plsc.subcore_barrier()

        def body(c, _):
            pltpu.sync_copy(idx_hbm.at[pl.ds(base + c * CH, CH)], idx_v)
            pltpu.sync_copy(ones_v, acc.at[idx_v], add=True)
            return _

        lax.fori_loop(0, NCHUNK, body, 0)
        plsc.subcore_barrier()
        pltpu.sync_copy(acc.at[pl.ds(sid * NPS, NPS)],
                        out_hbm.at[pl.ds(cid * N + sid * NPS, NPS)])

    return k(idx, jnp.ones((CH, W), f32), jnp.zeros((N, W), f32))


def _sc_gcn_pass(table, src, dst):
    W = TH
    CH = 200
    NCHUNK = EPW // CH

    @functools.partial(
        pl.kernel, mesh=_MESH,
        compiler_params=pltpu.CompilerParams(use_tc_tiling_on_sc=False),
        out_type=jax.ShapeDtypeStruct((NC * N, W), f32),
        scratch_types=[pltpu.VMEM((CH,), jnp.int32),
                       pltpu.VMEM((CH,), jnp.int32),
                       pltpu.VMEM((CH, W), f32),
                       pltpu.VMEM_SHARED((N, W), f32),
                       pltpu.SemaphoreType.DMA],
    )
    def k(table_hbm, src_hbm, dst_hbm, zero_hbm, out_hbm,
          sidx_v, didx_v, rows_v, acc, sem):
        cid = lax.axis_index("c")
        sid = lax.axis_index("s")
        wid = sid * NC + cid
        base = wid * EPW
        pltpu.sync_copy(zero_hbm.at[pl.ds(sid * NPS, NPS)],
                        acc.at[pl.ds(sid * NPS, NPS)])
        plsc.subcore_barrier()

        def body(c, _):
            off = base + c * CH
            pltpu.sync_copy(src_hbm.at[pl.ds(off, CH)], sidx_v)
            pltpu.sync_copy(dst_hbm.at[pl.ds(off, CH)], didx_v)
            pltpu.async_copy(table_hbm.at[sidx_v], rows_v, sem).wait()
            pltpu.sync_copy(rows_v, acc.at[didx_v], add=True)
            return _

        lax.fori_loop(0, NCHUNK, body, 0)
        plsc.subcore_barrier()
        pltpu.sync_copy(acc.at[pl.ds(sid * NPS, NPS)],
                        out_hbm.at[pl.ds(cid * N + sid * NPS, NPS)])

    return k(table, src, dst, jnp.zeros((N, W), f32))


def kernel(x, edge_attr, params, edge_index, batch):
    p = params
    src = edge_index[0]
    dst = edge_index[1]

    eye = jnp.eye(H, dtype=f32)
    expand = jnp.kron(eye, jnp.ones((1, H), f32))
    fold = jnp.tile(eye, (H, 1))

    ew, ewsc = _edge_net(edge_attr, p['en1_W'], p['en1_b'],
                         p['en2_W'], p['en2_b'])

    h = _node_proj(x, p['proj_W'], p['proj_b'])
    wi = [p['gru_Wih'][i * H:(i + 1) * H].T for i in range(3)]
    bi = [p['gru_bih'][i * H:(i + 1) * H].reshape(1, H) for i in range(3)]
    wh = [p['gru_Whh'][i * H:(i + 1) * H].T for i in range(3)]
    bh = [p['gru_bhh'][i * H:(i + 1) * H].reshape(1, H) for i in range(3)]

    deg = _sc_degree(dst)

    for _ in range(STEPS):
        ga = _sc_gather(h, src, 0, EA // NW)
        gb = _sc_gather(h, src, EA, EB // NW)
        ma = _msg(ga, ew, ewsc, expand, fold, 0)
        sa = _sc_scatter_add(ma, dst, H, 1000, 0, EA // NW)
        mb = _msg(gb, ew, ewsc, expand, fold, EA)
        sb = _sc_scatter_add(mb, dst, H, 1000, EA, EB // NW)
        h = _gru(sa[:N], sa[N:], sb[:N], sb[N:], h,
                 p['root'], p['nn_bias'], wi, bi, wh, bh)

    xg, hk, hv = _gcnprep(h, deg[:N], deg[N:], p['lin1_W'], p['lin1_b'],
                          p['p1k_W'], p['p1v_W'])
    pk = _sc_gcn_pass(hk, src, dst)
    pv = _sc_gcn_pass(hv, src, dst)
    kx, vx = _gcnfin(pk[:N], pk[N:], pv[:N], pv[N:], hk, hv, deg[:N], deg[N:],
                     p['p1k_b'], p['p1v_b'])

    s1p = jnp.pad(p['S1'][0], ((0, SP - S), (0, 0)))
    qp = _qp(s1p, p['p1q_W'], p['p1q_b'])

    qsl = qp[:S]
    hmask = jnp.repeat(jnp.eye(HEADS, dtype=f32), HD, axis=0)
    qb = (qsl.T[None, :, :] * hmask.T[:, :, None]).reshape(HEADS * TH, S)
    qb = jnp.pad(qb, ((0, 0), (0, SP - S)))
    sel = jnp.concatenate(
        [jnp.eye(TH, dtype=f32)[:, h * HD:(h + 1) * HD] for h in range(HEADS)],
        axis=0)
    selt = jnp.eye(TH, dtype=f32)
    r80 = jnp.kron(jnp.eye(SP, dtype=f32), jnp.ones((1, HD), f32))
    t16 = jnp.tile(jnp.eye(HD, dtype=f32), (1, SP))

    bat3 = batch.reshape(N // NT, 1, NT)
    pooled = _pool(kx, vx, bat3, qb, sel, r80, t16)

    pooled = pooled.reshape(B, HEADS, SP, HD).transpose(0, 2, 1, 3)
    poolbsd = pooled.reshape(B * SP, TH)

    out = _tail(poolbsd, qp, sel, selt, p)
    return out[:, :1]
